# baseline (speedup 1.0000x reference)
import functools

import jax
import jax.numpy as jnp
from jax import lax
from jax.experimental import pallas as pl
from jax.experimental.pallas import tpu as pltpu
from jax.experimental.pallas import tpu_sc as plsc

VOCAB = 1000000
MAXLEN = 200
EMBED = 64
BATCH = 1024

B = BATCH * MAXLEN
NC, NS = 2, 16
NW = NC * NS
BPW = B // NW
CHUNK = 128
NCHUNK = BPW // CHUNK
LROW = 128
VPR = EMBED // 16


@functools.partial(
    pl.kernel,
    mesh=plsc.VectorSubcoreMesh(core_axis_name="c", subcore_axis_name="s"),
    out_type=jax.ShapeDtypeStruct((B, EMBED), jnp.float32),
    scratch_types=[
        pltpu.VMEM((CHUNK,), jnp.int32),
        pltpu.VMEM((CHUNK,), jnp.int32),
        pltpu.VMEM((CHUNK, LROW), jnp.float32),
        pltpu.VMEM((CHUNK, EMBED), jnp.float32),
        pltpu.VMEM((MAXLEN, EMBED), jnp.float32),
        pltpu.SemaphoreType.DMA,
    ],
    compiler_params=pltpu.CompilerParams(use_tc_tiling_on_sc=True),
)
def _embed_sc(row_hbm, col_hbm, tok_hbm, pos_hbm, out_hbm,
              idx_v, col_v, big_v, out_v, pos_v, sem):
    wid = lax.axis_index("s") * NC + lax.axis_index("c")
    base = wid * BPW

    pltpu.sync_copy(pos_hbm, pos_v)

    def chunk_body(g, carry):
        start = base + g * CHUNK
        pltpu.sync_copy(row_hbm.at[pl.ds(start, CHUNK)], idx_v)
        pltpu.sync_copy(col_hbm.at[pl.ds(start, CHUNK)], col_v)
        pltpu.async_copy(tok_hbm.at[idx_v], big_v, sem).wait()

        off = lax.rem(start, MAXLEN)

        def grp_body(rg, c2):
            cb_vec = col_v[pl.ds(rg * 16, 16)]
            for j in range(16):
                r = rg * 16 + j
                cb = cb_vec[j]
                p = lax.rem(off + r, MAXLEN)
                for c in range(VPR):
                    sl = pl.ds(c * 16, 16)
                    out_v[r, sl] = (big_v[r, pl.ds(cb + c * 16, 16)]
                                    + pos_v[p, sl])
            return c2

        lax.fori_loop(0, CHUNK // 16, grp_body, 0)
        pltpu.sync_copy(out_v, out_hbm.at[pl.ds(start, CHUNK)])
        return carry

    lax.fori_loop(0, NCHUNK, chunk_body, 0)


def kernel(x, token_table, pos_table):
    xf = x.reshape(B).astype(jnp.int32)
    rows = xf >> 1
    cols = (xf & 1) << 6
    tok2 = token_table.reshape(VOCAB // 2, LROW)
    out = _embed_sc(rows, cols, tok2, pos_table)
    return out.reshape(BATCH, MAXLEN, EMBED)


# speedup vs baseline: 1.0031x; 1.0031x over previous
import functools

import jax
import jax.numpy as jnp
from jax import lax
from jax.experimental import pallas as pl
from jax.experimental.pallas import tpu as pltpu
from jax.experimental.pallas import tpu_sc as plsc

VOCAB = 1000000
MAXLEN = 200
EMBED = 64
BATCH = 1024

NC, NS = 2, 16
NW = NC * NS
SPW = BATCH // NW
B = BATCH * MAXLEN
NG = 13
RPAD = NG * 16
LAG = 4
VPR = EMBED // 16


@functools.partial(
    pl.kernel,
    mesh=plsc.VectorSubcoreMesh(core_axis_name="c", subcore_axis_name="s"),
    out_type=jax.ShapeDtypeStruct((BATCH, MAXLEN, EMBED), jnp.float32),
    scratch_types=[
        pltpu.VMEM((RPAD,), jnp.int32),
        pltpu.VMEM((2, RPAD, EMBED), jnp.float32),
        pltpu.VMEM((RPAD, EMBED), jnp.float32),
        pltpu.SemaphoreType.DMA,
        pltpu.SemaphoreType.DMA,
        pltpu.SemaphoreType.DMA,
        pltpu.SemaphoreType.DMA,
        pltpu.SemaphoreType.DMA,
        pltpu.SemaphoreType.DMA,
    ],
    compiler_params=pltpu.CompilerParams(use_tc_tiling_on_sc=True),
)
def _embed_sc(x_hbm, tok_hbm, pos_hbm, out_hbm,
              idx_v, bufs, pos_v, sr0, sr1, sr2, sr3, so0, so1):
    wid = lax.axis_index("s") * NC + lax.axis_index("c")
    b0 = wid * SPW
    rsems = (sr0, sr1, sr2, sr3)

    pltpu.sync_copy(pos_hbm, pos_v.at[pl.ds(0, MAXLEN), :])

    def out_drain(osem):
        pltpu.make_async_copy(
            out_hbm.at[0], bufs.at[0, pl.ds(0, MAXLEN), :], osem).wait()

    def seq_body(s, carry):
        b = b0 + s
        p = lax.rem(s, 2)
        even = p == 0

        @pl.when(jnp.logical_and(s >= 2, even))
        def _():
            out_drain(so0)

        @pl.when(jnp.logical_and(s >= 2, jnp.logical_not(even)))
        def _():
            out_drain(so1)

        pltpu.sync_copy(x_hbm.at[pl.ds(b * MAXLEN, MAXLEN)],
                        idx_v.at[pl.ds(0, MAXLEN)])
        t16 = idx_v[pl.ds(192, 16)]
        lane = lax.iota(jnp.int32, 16)
        idx_v[pl.ds(192, 16)] = jnp.where(lane < 8, t16, 0)

        def fire(g):
            v16 = idx_v[pl.ds(g * 16, 16)]
            for j in range(16):
                r = g * 16 + j
                pltpu.async_copy(
                    tok_hbm.at[pl.ds(v16[j], 1), :],
                    bufs.at[p, pl.ds(r, 1), :], rsems[g % LAG])

        def add(g):
            pltpu.make_async_copy(
                tok_hbm.at[pl.ds(0, 16), :],
                bufs.at[0, pl.ds(g * 16, 16), :], rsems[g % LAG]).wait()
            for j in range(16):
                r = g * 16 + j
                for c in range(VPR):
                    sl = pl.ds(c * 16, 16)
                    plsc.addupdate(bufs.at[p, r, sl], pos_v[r, sl])

        for g in range(LAG):
            fire(g)
        for g in range(LAG, NG):
            add(g - LAG)
            fire(g)
        for g in range(NG - LAG, NG):
            add(g)

        src = bufs.at[p, pl.ds(0, MAXLEN), :]

        @pl.when(even)
        def _():
            pltpu.async_copy(src, out_hbm.at[b], so0)

        @pl.when(jnp.logical_not(even))
        def _():
            pltpu.async_copy(src, out_hbm.at[b], so1)

        return carry

    lax.fori_loop(0, SPW, seq_body, 0)
    out_drain(so0)
    out_drain(so1)


def kernel(x, token_table, pos_table):
    xf = x.reshape(B).astype(jnp.int32)
    return _embed_sc(xf, token_table, pos_table)


# speedup vs baseline: 1.0826x; 1.0792x over previous
import functools

import jax
import jax.numpy as jnp
from jax import lax
from jax.experimental import pallas as pl
from jax.experimental.pallas import tpu as pltpu
from jax.experimental.pallas import tpu_sc as plsc

VOCAB = 1000000
MAXLEN = 200
EMBED = 64
BATCH = 1024

NC, NS = 2, 16
NW = NC * NS
SPW = BATCH // NW
B = BATCH * MAXLEN
G1 = 128
G2 = MAXLEN - G1
VPR = EMBED // 16


@functools.partial(
    pl.kernel,
    mesh=plsc.VectorSubcoreMesh(core_axis_name="c", subcore_axis_name="s"),
    out_type=jax.ShapeDtypeStruct((B, EMBED), jnp.float32),
    scratch_types=[
        pltpu.VMEM((2, MAXLEN), jnp.int32),
        pltpu.VMEM((2, MAXLEN, EMBED), jnp.float32),
        pltpu.VMEM((MAXLEN, EMBED), jnp.float32),
        pltpu.SemaphoreType.DMA,
        pltpu.SemaphoreType.DMA,
        pltpu.SemaphoreType.DMA,
        pltpu.SemaphoreType.DMA,
    ],
    compiler_params=pltpu.CompilerParams(use_tc_tiling_on_sc=False),
)
def _embed_sc(x_hbm, tok_hbm, pos_hbm, out_hbm,
              idx_v, bufs, pos_v, sg0, sg1, so0, so1):
    wid = lax.axis_index("s") * NC + lax.axis_index("c")
    b0 = wid * SPW

    pltpu.sync_copy(pos_hbm, pos_v)

    def out_drain(osem):
        pltpu.make_async_copy(
            out_hbm.at[pl.ds(0, MAXLEN), :],
            bufs.at[0, pl.ds(0, MAXLEN), :], osem).wait()

    def gather_drain(gsem):
        pltpu.make_async_copy(
            tok_hbm.at[pl.ds(0, MAXLEN), :],
            bufs.at[0, pl.ds(0, MAXLEN), :], gsem).wait()

    def fire(p, gsem):
        pltpu.async_copy(tok_hbm.at[idx_v.at[p, pl.ds(0, G1)]],
                         bufs.at[p, pl.ds(0, G1), :], gsem)
        pltpu.async_copy(tok_hbm.at[idx_v.at[p, pl.ds(G1, G2)]],
                         bufs.at[p, pl.ds(G1, G2), :], gsem)

    def step(s, carry):
        p = lax.rem(s, 2)
        q = 1 - p
        even = p == 0
        live = s < SPW

        @pl.when(jnp.logical_and(live, jnp.logical_and(s >= 2, even)))
        def _():
            out_drain(so0)

        @pl.when(jnp.logical_and(live,
                                 jnp.logical_and(s >= 2,
                                                 jnp.logical_not(even))))
        def _():
            out_drain(so1)

        @pl.when(live)
        def _():
            b = b0 + s
            pltpu.sync_copy(x_hbm.at[pl.ds(b * MAXLEN, MAXLEN)],
                            idx_v.at[p])

        @pl.when(jnp.logical_and(live, even))
        def _():
            fire(p, sg0)

        @pl.when(jnp.logical_and(live, jnp.logical_not(even)))
        def _():
            fire(p, sg1)

        @pl.when(jnp.logical_and(s >= 1, even))
        def _():
            gather_drain(sg1)

        @pl.when(jnp.logical_and(s >= 1, jnp.logical_not(even)))
        def _():
            gather_drain(sg0)

        @pl.when(s >= 1)
        def _():
            for r in range(MAXLEN):
                for c in range(VPR):
                    sl = pl.ds(c * 16, 16)
                    plsc.addupdate(bufs.at[q, r, sl], pos_v[r, sl])

        @pl.when(jnp.logical_and(s >= 1, even))
        def _():
            pltpu.async_copy(bufs.at[q, pl.ds(0, MAXLEN), :],
                             out_hbm.at[pl.ds((b0 + s - 1) * MAXLEN, MAXLEN), :],
                             so1)

        @pl.when(jnp.logical_and(s >= 1, jnp.logical_not(even)))
        def _():
            pltpu.async_copy(bufs.at[q, pl.ds(0, MAXLEN), :],
                             out_hbm.at[pl.ds((b0 + s - 1) * MAXLEN, MAXLEN), :],
                             so0)

        return carry

    lax.fori_loop(0, SPW + 1, step, 0)
    out_drain(so0)
    out_drain(so1)


def kernel(x, token_table, pos_table):
    xf = x.reshape(B).astype(jnp.int32)
    out = _embed_sc(xf, token_table, pos_table)
    return out.reshape(BATCH, MAXLEN, EMBED)


# speedup vs baseline: 1.0842x; 1.0015x over previous
import functools

import jax
import jax.numpy as jnp
from jax import lax
from jax.experimental import pallas as pl
from jax.experimental.pallas import tpu as pltpu
from jax.experimental.pallas import tpu_sc as plsc

VOCAB = 1000000
MAXLEN = 200
EMBED = 64
BATCH = 1024

NC, NS = 2, 16
NW = NC * NS
SPW = BATCH // NW
B = BATCH * MAXLEN
G1 = 128
G2 = MAXLEN - G1
VPR = EMBED // 16
NB = 3


@functools.partial(
    pl.kernel,
    mesh=plsc.VectorSubcoreMesh(core_axis_name="c", subcore_axis_name="s"),
    out_type=jax.ShapeDtypeStruct((B, EMBED), jnp.float32),
    scratch_types=[
        pltpu.VMEM((NB, MAXLEN), jnp.int32),
        pltpu.VMEM((NB, MAXLEN, EMBED), jnp.float32),
        pltpu.VMEM((MAXLEN, EMBED), jnp.float32),
        pltpu.SemaphoreType.DMA,
        pltpu.SemaphoreType.DMA,
        pltpu.SemaphoreType.DMA,
        pltpu.SemaphoreType.DMA,
        pltpu.SemaphoreType.DMA,
        pltpu.SemaphoreType.DMA,
    ],
    compiler_params=pltpu.CompilerParams(use_tc_tiling_on_sc=False),
)
def _embed_sc(x_hbm, tok_hbm, pos_hbm, out_hbm,
              idx_v, bufs, pos_v, sg0, sg1, sg2, so0, so1, so2):
    wid = lax.axis_index("s") * NC + lax.axis_index("c")
    b0 = wid * SPW
    gsems = (sg0, sg1, sg2)
    osems = (so0, so1, so2)

    pltpu.sync_copy(pos_hbm, pos_v)

    def out_drain(osem):
        pltpu.make_async_copy(
            out_hbm.at[pl.ds(0, MAXLEN), :],
            bufs.at[0, pl.ds(0, MAXLEN), :], osem).wait()

    def gather_drain(gsem):
        pltpu.make_async_copy(
            tok_hbm.at[pl.ds(0, MAXLEN), :],
            bufs.at[0, pl.ds(0, MAXLEN), :], gsem).wait()

    def fire(p, gsem):
        pltpu.async_copy(tok_hbm.at[idx_v.at[p, pl.ds(0, G1)]],
                         bufs.at[p, pl.ds(0, G1), :], gsem)
        pltpu.async_copy(tok_hbm.at[idx_v.at[p, pl.ds(G1, G2)]],
                         bufs.at[p, pl.ds(G1, G2), :], gsem)

    def step(s, carry):
        p = lax.rem(s, NB)
        f = lax.rem(s + 1, NB)
        live = s < SPW
        fin = jnp.logical_and(s >= 2, True)

        for i in range(NB):
            @pl.when(jnp.logical_and(s >= NB, p == i))
            def _(i=i):
                out_drain(osems[i])

        @pl.when(live)
        def _():
            b = b0 + s
            pltpu.sync_copy(x_hbm.at[pl.ds(b * MAXLEN, MAXLEN)],
                            idx_v.at[p])

        for i in range(NB):
            @pl.when(jnp.logical_and(live, p == i))
            def _(i=i):
                fire(p, gsems[i])

        for i in range(NB):
            @pl.when(jnp.logical_and(fin, f == i))
            def _(i=i):
                gather_drain(gsems[i])

        @pl.when(fin)
        def _():
            for r in range(MAXLEN):
                for c in range(VPR):
                    sl = pl.ds(c * 16, 16)
                    plsc.addupdate(bufs.at[f, r, sl], pos_v[r, sl])

        for i in range(NB):
            @pl.when(jnp.logical_and(fin, f == i))
            def _(i=i):
                pltpu.async_copy(
                    bufs.at[f, pl.ds(0, MAXLEN), :],
                    out_hbm.at[pl.ds((b0 + s - 2) * MAXLEN, MAXLEN), :],
                    osems[i])

        return carry

    lax.fori_loop(0, SPW + 2, step, 0)
    out_drain(osems[(SPW - 1) % NB])


def kernel(x, token_table, pos_table):
    xf = x.reshape(B).astype(jnp.int32)
    out = _embed_sc(xf, token_table, pos_table)
    return out.reshape(BATCH, MAXLEN, EMBED)
